# baseline (speedup 1.0000x reference)
import dataclasses
import functools

import jax
import jax.numpy as jnp
from jax import lax
from jax.experimental import pallas as pl
from jax.experimental.pallas import tpu as pltpu
from jax.experimental.pallas import tpu_sc as plsc

_D = 64
_S = 200
_B = 1024
_NC = 2
_NS = 16
_NW = _NC * _NS
_G0 = 128
_G1 = _S - _G0
_NB = 4
_UP = 104


def _compiler_params():
    cp = pltpu.CompilerParams(use_tc_tiling_on_sc=False)
    if "needs_layout_passes" in pltpu.CompilerParams.__dataclass_fields__:
        cp = dataclasses.replace(cp, needs_layout_passes=False)
    return cp


def _sc_gather(x, emb):
    spt = _B // _NW
    mesh = plsc.VectorSubcoreMesh(core_axis_name="c", subcore_axis_name="s")

    @functools.partial(
        pl.kernel,
        mesh=mesh,
        compiler_params=_compiler_params(),
        out_type=jax.ShapeDtypeStruct((_B * 2 * _UP, _D), jnp.float32),
        scratch_types=[
            pltpu.VMEM((spt, _S), jnp.int32),
            pltpu.VMEM((_NB, 4 * _UP, _D), jnp.float32),
            pltpu.SemaphoreType.DMA((_NB,)),
            pltpu.SemaphoreType.DMA((_NB,)),
        ],
    )
    def k(emb_hbm, x_hbm, y_hbm, idx_all, rows, gsem, osem):
        wid = lax.axis_index("s") * _NC + lax.axis_index("c")
        seq0 = wid * spt
        pltpu.sync_copy(x_hbm.at[pl.ds(seq0, spt)], idx_all)

        def start_gather(cloc, j):
            for t in range(2):
                base = t * 2 * _UP
                seq = 2 * cloc + t
                pltpu.async_copy(emb_hbm.at[idx_all.at[seq, pl.ds(0, _G0)]],
                                 rows.at[j].at[pl.ds(base, _G0)], gsem.at[j])
                pltpu.async_copy(emb_hbm.at[idx_all.at[seq, pl.ds(_G0, _G1)]],
                                 rows.at[j].at[pl.ds(base + _G0, _G1)],
                                 gsem.at[j])

        for j in range(_NB):
            start_gather(j, j)

        nch = spt // 2

        @pl.loop(0, nch, step=_NB)
        def _(c):
            for j in range(_NB):
                cloc = c + j
                for t in range(2):
                    pltpu.make_async_copy(
                        emb_hbm.at[pl.ds(0, _S)],
                        rows.at[j].at[pl.ds(t * 2 * _UP, _S)],
                        gsem.at[j]).wait()
                pltpu.async_copy(
                    rows.at[j],
                    y_hbm.at[pl.ds((seq0 + 2 * cloc) * 2 * _UP, 4 * _UP)],
                    osem.at[j])

                @pl.when(cloc + _NB < nch)
                def _():
                    pltpu.make_async_copy(rows.at[j],
                                          y_hbm.at[pl.ds(0, 4 * _UP)],
                                          osem.at[j]).wait()
                    start_gather(cloc + _NB, j)

        for j in range(_NB):
            pltpu.make_async_copy(rows.at[j], y_hbm.at[pl.ds(0, 4 * _UP)],
                                  osem.at[j]).wait()

    return k(emb, x)


def _tc_linearize(emb):
    def body(x_ref, o_ref):
        o_ref[...] = x_ref[...].reshape(o_ref.shape)

    v = emb.shape[0]
    blk = 2000
    return pl.pallas_call(
        body,
        grid=(v // blk,),
        in_specs=[pl.BlockSpec((blk, _D), lambda i: (i, 0))],
        out_specs=pl.BlockSpec((blk // 2, 2 * _D), lambda i: (i, 0)),
        out_shape=jax.ShapeDtypeStruct((v // 2, 2 * _D), jnp.float32),
    )(emb)


def _tc_relayout(y3, posr):
    def body(y_ref, p_ref, o_ref):
        for u in range(_S // 2):
            xb = y_ref[:, u, :] + p_ref[u, :]
            o_ref[pl.ds(2 * u, 2)] = xb.T.reshape(2, 8, 1, 8, 128)

    return pl.pallas_call(
        body,
        grid=(_B // 128,),
        in_specs=[
            pl.BlockSpec((128, _UP, 128), lambda tc: (tc, 0, 0)),
            pl.BlockSpec((_S // 2, 128), lambda tc: (0, 0)),
        ],
        out_specs=pl.BlockSpec((_S, _D // 8, 1, 8, 128),
                               lambda tc: (0, 0, tc, 0, 0)),
        out_shape=jax.ShapeDtypeStruct((_S, _D // 8, _B // 128, 8, 128),
                                       jnp.float32),
    )(y3, posr)


def kernel(x, embedding, pos_embedding):
    emb_packed = jnp.concatenate(
        [embedding[0::2, :], embedding[1::2, :]], axis=1)
    emb_lin = emb_packed.reshape(embedding.shape)
    y = _sc_gather(x.astype(jnp.int32), emb_lin)
    y3 = y.reshape(_B, _UP, 2 * _D)
    posr = pos_embedding.reshape(_S // 2, 2 * _D)
    out5 = _tc_relayout(y3, posr)
    return out5.transpose(2, 4, 0, 1, 3).reshape(_B, _S, _D)


# speedup vs baseline: 6.0949x; 6.0949x over previous
import dataclasses
import functools

import jax
import jax.numpy as jnp
from jax import lax
from jax.experimental import pallas as pl
from jax.experimental.pallas import tpu as pltpu
from jax.experimental.pallas import tpu_sc as plsc

_D = 64
_S = 200
_B = 1024
_NC = 2
_NS = 16
_NW = _NC * _NS
_G0 = 128
_G1 = _S - _G0
_NB = 4
_UP = 104


def _compiler_params():
    cp = pltpu.CompilerParams(use_tc_tiling_on_sc=False)
    if "needs_layout_passes" in pltpu.CompilerParams.__dataclass_fields__:
        cp = dataclasses.replace(cp, needs_layout_passes=False)
    return cp


def _sc_gather(x, emb):
    spt = _B // _NW
    mesh = plsc.VectorSubcoreMesh(core_axis_name="c", subcore_axis_name="s")

    @functools.partial(
        pl.kernel,
        mesh=mesh,
        compiler_params=_compiler_params(),
        out_type=jax.ShapeDtypeStruct((_B * 2 * _UP, _D), jnp.float32),
        scratch_types=[
            pltpu.VMEM((spt, _S), jnp.int32),
            pltpu.VMEM((_NB, 4 * _UP, _D), jnp.float32),
            pltpu.SemaphoreType.DMA((_NB,)),
            pltpu.SemaphoreType.DMA((_NB,)),
        ],
    )
    def k(emb_hbm, x_hbm, y_hbm, idx_all, rows, gsem, osem):
        wid = lax.axis_index("s") * _NC + lax.axis_index("c")
        seq0 = wid * spt
        pltpu.sync_copy(x_hbm.at[pl.ds(seq0, spt)], idx_all)

        def start_gather(cloc, j):
            for t in range(2):
                base = t * 2 * _UP
                seq = 2 * cloc + t
                pltpu.async_copy(emb_hbm.at[idx_all.at[seq, pl.ds(0, _G0)]],
                                 rows.at[j].at[pl.ds(base, _G0)], gsem.at[j])
                pltpu.async_copy(emb_hbm.at[idx_all.at[seq, pl.ds(_G0, _G1)]],
                                 rows.at[j].at[pl.ds(base + _G0, _G1)],
                                 gsem.at[j])

        for j in range(_NB):
            start_gather(j, j)

        nch = spt // 2

        @pl.loop(0, nch, step=_NB)
        def _(c):
            for j in range(_NB):
                cloc = c + j
                for t in range(2):
                    pltpu.make_async_copy(
                        emb_hbm.at[pl.ds(0, _S)],
                        rows.at[j].at[pl.ds(t * 2 * _UP, _S)],
                        gsem.at[j]).wait()
                pltpu.async_copy(
                    rows.at[j],
                    y_hbm.at[pl.ds((seq0 + 2 * cloc) * 2 * _UP, 4 * _UP)],
                    osem.at[j])

                @pl.when(cloc + _NB < nch)
                def _():
                    pltpu.make_async_copy(rows.at[j],
                                          y_hbm.at[pl.ds(0, 4 * _UP)],
                                          osem.at[j]).wait()
                    start_gather(cloc + _NB, j)

        for j in range(_NB):
            pltpu.make_async_copy(rows.at[j], y_hbm.at[pl.ds(0, 4 * _UP)],
                                  osem.at[j]).wait()

    return k(emb, x)


def _tc_linearize(emb):
    def body(x_ref, o_ref):
        o_ref[...] = x_ref[...].reshape(o_ref.shape)

    v = emb.shape[0]
    blk = 2000
    return pl.pallas_call(
        body,
        grid=(v // blk,),
        in_specs=[pl.BlockSpec((blk, _D), lambda i: (i, 0))],
        out_specs=pl.BlockSpec((blk // 2, 2 * _D), lambda i: (i, 0)),
        out_shape=jax.ShapeDtypeStruct((v // 2, 2 * _D), jnp.float32),
    )(emb)


def _tc_relayout(y3, posr):
    def body(y_ref, p_ref, o_ref):
        for u in range(_S // 2):
            xb = y_ref[:, u, :] + p_ref[u, :]
            o_ref[pl.ds(2 * u, 2)] = xb.T.reshape(2, 8, 1, 8, 128)

    return pl.pallas_call(
        body,
        grid=(_B // 128,),
        in_specs=[
            pl.BlockSpec((128, _UP, 128), lambda tc: (tc, 0, 0)),
            pl.BlockSpec((_S // 2, 128), lambda tc: (0, 0)),
        ],
        out_specs=pl.BlockSpec((_S, _D // 8, 1, 8, 128),
                               lambda tc: (0, 0, tc, 0, 0)),
        out_shape=jax.ShapeDtypeStruct((_S, _D // 8, _B // 128, 8, 128),
                                       jnp.float32),
    )(y3, posr)


def kernel(x, embedding, pos_embedding):
    emb_packed = jax.lax.optimization_barrier(
        embedding.reshape(embedding.shape[0] // 2, 2 * _D))
    emb_lin = emb_packed.reshape(embedding.shape)
    y = _sc_gather(x.astype(jnp.int32), emb_lin)
    y3 = y.reshape(_B, _UP, 2 * _D)
    posr = pos_embedding.reshape(_S // 2, 2 * _D)
    out5 = _tc_relayout(y3, posr)
    return out5.transpose(2, 4, 0, 1, 3).reshape(_B, _S, _D)


# speedup vs baseline: 6.1448x; 1.0082x over previous
import dataclasses
import functools

import jax
import jax.numpy as jnp
from jax import lax
from jax.experimental import pallas as pl
from jax.experimental.pallas import tpu as pltpu
from jax.experimental.pallas import tpu_sc as plsc

_D = 64
_S = 200
_B = 1024
_NC = 2
_NS = 16
_NW = _NC * _NS
_G0 = 128
_G1 = _S - _G0
_NB = 4
_UP = 104


def _compiler_params():
    cp = pltpu.CompilerParams(use_tc_tiling_on_sc=False)
    if "needs_layout_passes" in pltpu.CompilerParams.__dataclass_fields__:
        cp = dataclasses.replace(cp, needs_layout_passes=False)
    return cp


def _sc_gather(x, emb):
    nseq = x.shape[0]
    spt = nseq // _NW
    mesh = plsc.VectorSubcoreMesh(core_axis_name="c", subcore_axis_name="s")

    @functools.partial(
        pl.kernel,
        mesh=mesh,
        compiler_params=_compiler_params(),
        out_type=jax.ShapeDtypeStruct((nseq * 2 * _UP, _D), jnp.float32),
        scratch_types=[
            pltpu.VMEM((spt, _S), jnp.int32),
            pltpu.VMEM((_NB, 4 * _UP, _D), jnp.float32),
            pltpu.SemaphoreType.DMA((_NB,)),
            pltpu.SemaphoreType.DMA((_NB,)),
        ],
    )
    def k(emb_hbm, x_hbm, y_hbm, idx_all, rows, gsem, osem):
        wid = lax.axis_index("s") * _NC + lax.axis_index("c")
        seq0 = wid * spt
        pltpu.sync_copy(x_hbm.at[pl.ds(seq0, spt)], idx_all)

        def start_gather(cloc, j):
            for t in range(2):
                base = t * 2 * _UP
                seq = 2 * cloc + t
                pltpu.async_copy(emb_hbm.at[idx_all.at[seq, pl.ds(0, _G0)]],
                                 rows.at[j].at[pl.ds(base, _G0)], gsem.at[j])
                pltpu.async_copy(emb_hbm.at[idx_all.at[seq, pl.ds(_G0, _G1)]],
                                 rows.at[j].at[pl.ds(base + _G0, _G1)],
                                 gsem.at[j])

        for j in range(_NB):
            start_gather(j, j)

        nch = spt // 2

        @pl.loop(0, nch, step=_NB)
        def _(c):
            for j in range(_NB):
                cloc = c + j
                for t in range(2):
                    pltpu.make_async_copy(
                        emb_hbm.at[pl.ds(0, _S)],
                        rows.at[j].at[pl.ds(t * 2 * _UP, _S)],
                        gsem.at[j]).wait()
                pltpu.async_copy(
                    rows.at[j],
                    y_hbm.at[pl.ds((seq0 + 2 * cloc) * 2 * _UP, 4 * _UP)],
                    osem.at[j])

                @pl.when(cloc + _NB < nch)
                def _():
                    pltpu.make_async_copy(rows.at[j],
                                          y_hbm.at[pl.ds(0, 4 * _UP)],
                                          osem.at[j]).wait()
                    start_gather(cloc + _NB, j)

        for j in range(_NB):
            pltpu.make_async_copy(rows.at[j], y_hbm.at[pl.ds(0, 4 * _UP)],
                                  osem.at[j]).wait()

    return k(emb, x)


def _tc_linearize(emb):
    def body(x_ref, o_ref):
        o_ref[...] = x_ref[...].reshape(o_ref.shape)

    v = emb.shape[0]
    blk = 2000
    return pl.pallas_call(
        body,
        grid=(v // blk,),
        in_specs=[pl.BlockSpec((blk, _D), lambda i: (i, 0))],
        out_specs=pl.BlockSpec((blk // 2, 2 * _D), lambda i: (i, 0)),
        out_shape=jax.ShapeDtypeStruct((v // 2, 2 * _D), jnp.float32),
    )(emb)


def _tc_relayout(y3, posr, off, prev=None):
    nblk = y3.shape[0] // 128

    def body(*refs):
        y_ref, p_ref, o_ref = refs[-3], refs[-2], refs[-1]
        for u in range(_S // 2):
            xb = y_ref[:, u, :] + p_ref[u, :]
            o_ref[pl.ds(2 * u, 2)] = xb.T.reshape(2, 8, 1, 8, 128)

    in_specs = [
        pl.BlockSpec((128, _UP, 128), lambda tc: (tc, 0, 0)),
        pl.BlockSpec((_S // 2, 128), lambda tc: (0, 0)),
    ]
    args = (y3, posr)
    kwargs = {}
    if prev is not None:
        in_specs = [pl.BlockSpec(memory_space=pl.ANY)] + in_specs
        args = (prev, y3, posr)
        kwargs = dict(input_output_aliases={0: 0})
    return pl.pallas_call(
        body,
        grid=(nblk,),
        in_specs=in_specs,
        out_specs=pl.BlockSpec((_S, _D // 8, 1, 8, 128),
                               lambda tc: (0, 0, tc + off, 0, 0)),
        out_shape=jax.ShapeDtypeStruct((_S, _D // 8, _B // 128, 8, 128),
                                       jnp.float32),
        **kwargs,
    )(*args)


def kernel(x, embedding, pos_embedding):
    emb_packed = jax.lax.optimization_barrier(
        embedding.reshape(embedding.shape[0] // 2, 2 * _D))
    emb_lin = emb_packed.reshape(embedding.shape)
    xi = x.astype(jnp.int32)
    posr = pos_embedding.reshape(_S // 2, 2 * _D)
    half = _B // 2
    ya = _sc_gather(xi[:half], emb_lin).reshape(half, _UP, 2 * _D)
    yb = _sc_gather(xi[half:], emb_lin).reshape(half, _UP, 2 * _D)
    out5a = _tc_relayout(ya, posr, 0)
    out5 = _tc_relayout(yb, posr, half // 128, prev=out5a)
    return out5.transpose(2, 4, 0, 1, 3).reshape(_B, _S, _D)
